# baseline (speedup 1.0000x reference)
import functools

import jax
import jax.numpy as jnp
from jax.experimental import pallas as pl


def _gap_kernel(x_ref, o_ref, *, nl, scale):
    l = pl.program_id(2)
    s = jnp.sum(x_ref[0], axis=1)

    @pl.when(l == 0)
    def _init():
        o_ref[0, 0] = s

    @pl.when(l > 0)
    def _acc():
        o_ref[0, 0] = o_ref[0, 0] + s

    @pl.when(l == nl - 1)
    def _scale():
        o_ref[0, 0] = o_ref[0, 0] * scale


def _head_kernel(gap_ref, w1_ref, b1_ref, w2_ref, b2_ref,
                 ev_ref, ei_ref, loss_ref, *, e, k, eps):
    gap = gap_ref[...]
    h = jax.lax.dot_general(
        gap, w1_ref[...], (((1,), (1,)), ((), ())),
        preferred_element_type=jnp.float32)
    h = jax.nn.relu(h + b1_ref[...][None, :])
    pre = jax.lax.dot_general(
        h, w2_ref[...], (((1,), (1,)), ((), ())),
        preferred_element_type=jnp.float32)
    pre = pre + b2_ref[...][None, :]

    m = jnp.max(pre, axis=1, keepdims=True)
    ex = jnp.exp(pre - m)
    logits = ex / jnp.sum(ex, axis=1, keepdims=True)

    b = logits.shape[0]
    ids = jax.lax.broadcasted_iota(jnp.int32, (b, e), 1)

    m1 = jnp.max(logits, axis=1, keepdims=True)
    i1 = jnp.min(jnp.where(logits == m1, ids, e), axis=1, keepdims=True)
    masked = jnp.where(ids == i1, -jnp.inf, logits)
    m2 = jnp.max(masked, axis=1, keepdims=True)
    i2 = jnp.min(jnp.where(masked == m2, ids, e), axis=1, keepdims=True)

    vals = jnp.concatenate([m1, m2], axis=1)
    if k < e:
        ev = jnp.exp(vals - m1)
        vals = ev / jnp.sum(ev, axis=1, keepdims=True)
    ev_ref[...] = vals
    ei_ref[...] = jnp.concatenate([i1, i2], axis=1).astype(jnp.int32)

    n = b * e
    mean = jnp.sum(logits) / n
    var = jnp.sum((logits - mean) ** 2) / (n - 1)
    loss_ref[...] = (jnp.sqrt(var) / (mean + eps)).reshape(1, 1)


def kernel(x, W1, b1, W2, b2):
    B, C, H, W = x.shape
    HID = W1.shape[0]
    E = W2.shape[0]
    K = 2
    EPS = 1e-10
    L = H * W
    xr = x.reshape(B, C, L)

    Cb = 256 if C % 256 == 0 else C
    nl = 1
    for cand in (8, 4, 2):
        if L % (cand * 128) == 0:
            nl = cand
            break
    Lb = L // nl

    gap = pl.pallas_call(
        functools.partial(_gap_kernel, nl=nl, scale=1.0 / L),
        grid=(B, C // Cb, nl),
        in_specs=[pl.BlockSpec((1, Cb, Lb), lambda b, c, l: (b, c, l))],
        out_specs=pl.BlockSpec((1, 1, Cb), lambda b, c, l: (b, 0, c)),
        out_shape=jax.ShapeDtypeStruct((B, 1, C), jnp.float32),
    )(xr).reshape(B, C)

    ev, ei, loss = pl.pallas_call(
        functools.partial(_head_kernel, e=E, k=K, eps=EPS),
        out_shape=(
            jax.ShapeDtypeStruct((B, K), jnp.float32),
            jax.ShapeDtypeStruct((B, K), jnp.int32),
            jax.ShapeDtypeStruct((1, 1), jnp.float32),
        ),
    )(gap, W1, b1, W2, b2)
    return ev, ei, loss[0, 0]
